# Initial kernel scaffold; baseline (speedup 1.0000x reference)
import jax
import jax.numpy as jnp
from jax.experimental import pallas as pl


def kernel(node, edge, edge_index, node_graph_ids, params):
    raise NotImplementedError("write your pallas kernel here")


# speedup vs baseline: 3.4991x; 3.4991x over previous
import functools

import jax
import jax.numpy as jnp
from jax import lax
from jax.experimental import pallas as pl
from jax.experimental.pallas import tpu as pltpu
from jax.experimental.pallas import tpu_sc as plsc

N = 50000
E = 800000
G = 2000
H = 64

N_PAD = 51200
E_PAD = 819200
G_PAD = 2048

_BN = 3200
_BE = 4096

_f32 = jnp.float32


def _lrelu(x):
    return jnp.where(x >= 0, x, 0.01 * x)


def _dot_t(x, w):
    return lax.dot_general(x, w, (((1,), (1,)), ((), ())),
                           preferred_element_type=_f32)


def _rowdot(x, a):
    return jnp.sum(x * a, axis=1, keepdims=True)


def _row_spec(bm, n):
    return pl.BlockSpec((bm, n), lambda i: (i, 0))


def _full_spec(a, b):
    return pl.BlockSpec((a, b), lambda i: (0, 0))


def _emb_body(x_ref, w_ref, b_ref, o_ref):
    o_ref[...] = _lrelu(_dot_t(x_ref[...], w_ref[...]) + b_ref[...])


def _tc_embed(x, w, b, bm):
    m, k = x.shape
    h = w.shape[0]
    return pl.pallas_call(
        _emb_body,
        grid=(m // bm,),
        in_specs=[_row_spec(bm, k), _full_spec(h, k), _full_spec(1, h)],
        out_specs=_row_spec(bm, h),
        out_shape=jax.ShapeDtypeStruct((m, h), _f32),
    )(x, w, b)


def _node_body(x_ref, wn_ref, bn_ref, w1_ref, a2_ref, h_ref, hw1_ref, hda_ref):
    h = _lrelu(_dot_t(x_ref[...], wn_ref[...]) + bn_ref[...])
    h_ref[...] = h
    hw1_ref[...] = _dot_t(h, w1_ref[...])
    hda_ref[...] = _rowdot(h, a2_ref[...])


def _tc_node(x, wn, bn, w1, a2):
    return pl.pallas_call(
        _node_body,
        grid=(N_PAD // _BN,),
        in_specs=[_row_spec(_BN, H), _full_spec(H, H), _full_spec(1, H),
                  _full_spec(H, H), _full_spec(1, H)],
        out_specs=[_row_spec(_BN, H), _row_spec(_BN, H), _row_spec(_BN, 1)],
        out_shape=[jax.ShapeDtypeStruct((N_PAD, H), _f32),
                   jax.ShapeDtypeStruct((N_PAD, H), _f32),
                   jax.ShapeDtypeStruct((N_PAD, 1), _f32)],
    )(x, wn, bn, w1, a2)


def _edge_body(hs_ref, e_ref, w2_ref, eb_ref, a1_ref, ab_ref, nm_ref, s1_ref):
    nm = _lrelu(hs_ref[...] + _dot_t(e_ref[...], w2_ref[...]) + eb_ref[...])
    nm_ref[...] = nm
    s1_ref[...] = _rowdot(nm, a1_ref[...]) + ab_ref[0, 0]


def _tc_edge(hsrc, e, w2, eb, a1, ab):
    return pl.pallas_call(
        _edge_body,
        grid=(E_PAD // _BE,),
        in_specs=[_row_spec(_BE, H), _row_spec(_BE, H), _full_spec(H, H),
                  _full_spec(1, H), _full_spec(1, H), _full_spec(1, 1)],
        out_specs=[_row_spec(_BE, H), _row_spec(_BE, 1)],
        out_shape=[jax.ShapeDtypeStruct((E_PAD, H), _f32),
                   jax.ShapeDtypeStruct((E_PAD, 1), _f32)],
    )(hsrc, e, w2, eb, a1, ab)


def _attc_body(nm_ref, att_ref, w_ref, b_ref, oa_ref, ob_ref):
    attn = _dot_t(nm_ref[...], w_ref[...]) + b_ref[...]
    attc = att_ref[...] * attn
    oa_ref[...] = attc[:, :H // 2]
    ob_ref[...] = attc[:, H // 2:]


def _tc_attc(nm, att, w, b, bm):
    m = nm.shape[0]
    return pl.pallas_call(
        _attc_body,
        grid=(m // bm,),
        in_specs=[_row_spec(bm, H), _row_spec(bm, 1), _full_spec(H, H),
                  _full_spec(1, H)],
        out_specs=[_row_spec(bm, H // 2), _row_spec(bm, H // 2)],
        out_shape=[jax.ShapeDtypeStruct((m, H // 2), _f32),
                   jax.ShapeDtypeStruct((m, H // 2), _f32)],
    )(nm, att, w, b)


def _gru_body(ca_ref, cb_ref, h_ref, wih_ref, whh_ref, bih_ref, bhh_ref,
              o_ref):
    ctx = jnp.concatenate([ca_ref[...], cb_ref[...]], axis=1)
    ctx = jnp.where(ctx > 0, ctx, jnp.exp(ctx) - 1.0)
    h = h_ref[...]
    gi = _dot_t(ctx, wih_ref[...]) + bih_ref[...]
    gh = _dot_t(h, whh_ref[...]) + bhh_ref[...]
    r = jax.nn.sigmoid(gi[:, :H] + gh[:, :H])
    z = jax.nn.sigmoid(gi[:, H:2 * H] + gh[:, H:2 * H])
    n = jnp.tanh(gi[:, 2 * H:] + r * gh[:, 2 * H:])
    o_ref[...] = jnp.maximum((1.0 - z) * n + z * h, 0.0)


def _tc_gru(ca, cb, h, g, bm):
    m = ca.shape[0]
    return pl.pallas_call(
        _gru_body,
        grid=(m // bm,),
        in_specs=[_row_spec(bm, H // 2), _row_spec(bm, H // 2),
                  _row_spec(bm, H), _full_spec(3 * H, H), _full_spec(3 * H, H),
                  _full_spec(1, 3 * H), _full_spec(1, 3 * H)],
        out_specs=_row_spec(bm, H),
        out_shape=jax.ShapeDtypeStruct((m, H), _f32),
    )(ca, cb, h, g['Wih'], g['Whh'], g['bih'].reshape(1, -1),
      g['bhh'].reshape(1, -1))


def _split_body(x_ref, oa_ref, ob_ref):
    x = x_ref[...]
    oa_ref[...] = x[:, :H // 2]
    ob_ref[...] = x[:, H // 2:]


def _tc_split(x):
    return pl.pallas_call(
        _split_body,
        grid=(N_PAD // _BN,),
        in_specs=[_row_spec(_BN, H)],
        out_specs=[_row_spec(_BN, H // 2), _row_spec(_BN, H // 2)],
        out_shape=[jax.ShapeDtypeStruct((N_PAD, H // 2), _f32),
                   jax.ShapeDtypeStruct((N_PAD, H // 2), _f32)],
    )(x)


def _molg_body(s_ref, a2_ref, sa_ref, ga_ref):
    sa = _lrelu(s_ref[...])
    sa_ref[...] = sa
    ga_ref[...] = _rowdot(sa, a2_ref[...])


def _tc_molg(s, a2):
    return pl.pallas_call(
        _molg_body,
        grid=(1,),
        in_specs=[_row_spec(G_PAD, H), _full_spec(1, H)],
        out_specs=[_row_spec(G_PAD, H), _row_spec(G_PAD, 1)],
        out_shape=[jax.ShapeDtypeStruct((G_PAD, H), _f32),
                   jax.ShapeDtypeStruct((G_PAD, 1), _f32)],
    )(s, a2)


def _moln_body(x_ref, a1_ref, ab_ref, w_ref, b_ref, s1_ref, hn_ref):
    x = x_ref[...]
    s1_ref[...] = _rowdot(x, a1_ref[...]) + ab_ref[0, 0]
    hn_ref[...] = _dot_t(x, w_ref[...]) + b_ref[...]


def _tc_moln(x, a1, ab, w, b):
    return pl.pallas_call(
        _moln_body,
        grid=(N_PAD // _BN,),
        in_specs=[_row_spec(_BN, H), _full_spec(1, H), _full_spec(1, 1),
                  _full_spec(H, H), _full_spec(1, H)],
        out_specs=[_row_spec(_BN, 1), _row_spec(_BN, H)],
        out_shape=[jax.ShapeDtypeStruct((N_PAD, 1), _f32),
                   jax.ShapeDtypeStruct((N_PAD, H), _f32)],
    )(x, a1, ab, w, b)


def _awhn_body(hn_ref, aw_ref, oa_ref, ob_ref):
    v = aw_ref[...] * hn_ref[...]
    oa_ref[...] = v[:, :H // 2]
    ob_ref[...] = v[:, H // 2:]


def _tc_awhn(hn, aw):
    return pl.pallas_call(
        _awhn_body,
        grid=(N_PAD // _BN,),
        in_specs=[_row_spec(_BN, H), _row_spec(_BN, 1)],
        out_specs=[_row_spec(_BN, H // 2), _row_spec(_BN, H // 2)],
        out_shape=[jax.ShapeDtypeStruct((N_PAD, H // 2), _f32),
                   jax.ShapeDtypeStruct((N_PAD, H // 2), _f32)],
    )(hn, aw)


def _pred_body(s_ref, w_ref, b_ref, o_ref):
    o_ref[...] = _rowdot(s_ref[...], w_ref[...]) + b_ref[0, 0]


def _tc_pred(s, w, b):
    return pl.pallas_call(
        _pred_body,
        grid=(1,),
        in_specs=[_row_spec(G_PAD, H), _full_spec(1, H), _full_spec(1, 1)],
        out_specs=_row_spec(G_PAD, 1),
        out_shape=jax.ShapeDtypeStruct((G_PAD, 1), _f32),
    )(s, w, b)


_MESH = plsc.VectorSubcoreMesh(core_axis_name="c", subcore_axis_name="s")
_SC_PARAMS = pltpu.CompilerParams(use_tc_tiling_on_sc=False,
                                  needs_layout_passes=False)
_NC = 2
_NS = 16
_NW = _NC * _NS
_C = 128


def _gather_rows(table, idx):
    per_w = E_PAD // _NW
    n_chunks = per_w // _C

    @functools.partial(
        pl.kernel,
        mesh=_MESH,
        out_type=jax.ShapeDtypeStruct((E_PAD, H), _f32),
        scratch_types=[
            pltpu.VMEM((_C,), jnp.int32),
            pltpu.VMEM((_C, H), _f32),
            pltpu.SemaphoreType.DMA,
        ],
        compiler_params=_SC_PARAMS,
    )
    def k(table_hbm, idx_hbm, out_hbm, idx_v, rows_v, sem):
        wid = lax.axis_index("s") * _NC + lax.axis_index("c")
        base = wid * per_w

        def body(i, carry):
            off = base + i * _C
            pltpu.sync_copy(idx_hbm.at[pl.ds(off, _C)], idx_v)
            pltpu.async_copy(table_hbm.at[idx_v], rows_v, sem).wait()
            pltpu.sync_copy(rows_v, out_hbm.at[pl.ds(off, _C)])
            return carry

        lax.fori_loop(0, n_chunks, body, 0)

    return k(table, idx)


def _seg_softmax_sc(s1, table, idx, zeros, m_sz, k_sz):
    per_sub = m_sz // _NS
    n_chunks = per_sub // _C

    @functools.partial(
        pl.kernel,
        mesh=_MESH,
        out_type=jax.ShapeDtypeStruct((m_sz,), _f32),
        scratch_types=[
            pltpu.VMEM_SHARED((k_sz,), _f32),
            pltpu.VMEM((k_sz,), _f32),
            pltpu.VMEM((k_sz,), _f32),
            pltpu.VMEM((1, _C), jnp.int32),
            pltpu.VMEM((_C,), _f32),
            pltpu.VMEM((_C,), _f32),
        ],
        compiler_params=_SC_PARAMS,
    )
    def k(s1_hbm, tab_hbm, idx_hbm, z_hbm, out_hbm,
          spsum, tab_v, sums_v, idx_v, s1_v, w_v):
        c = lax.axis_index("c")
        s = lax.axis_index("s")
        pltpu.sync_copy(tab_hbm, tab_v)

        @pl.when(s == 0)
        def _():
            pltpu.sync_copy(z_hbm, spsum)

        plsc.subcore_barrier()

        def _chunk_w(off):
            pltpu.sync_copy(idx_hbm.at[pl.ds(off, _C)], idx_v.at[0])
            pltpu.sync_copy(s1_hbm.at[pl.ds(off, _C)], s1_v)
            for j in range(_C // 16):
                sl = pl.ds(j * 16, 16)
                iv = idx_v[0, sl]
                tv = plsc.load_gather(tab_v, [iv])
                sc = s1_v[sl] + tv
                sc = jnp.where(sc >= 0, sc, 0.01 * sc)
                w_v[sl] = jnp.exp(sc)

        def ph1(i, carry):
            off = s * per_sub + i * _C
            _chunk_w(off)
            pltpu.sync_copy(w_v, spsum.at[idx_v.at[0]], add=True)
            return carry

        lax.fori_loop(0, n_chunks, ph1, 0)
        plsc.subcore_barrier()
        pltpu.sync_copy(spsum, sums_v)

        def ph2(i, carry):
            off = s * per_sub + i * _C
            _chunk_w(off)
            for j in range(_C // 16):
                sl = pl.ds(j * 16, 16)
                iv = idx_v[0, sl]
                sg = plsc.load_gather(sums_v, [iv])
                w_v[sl] = w_v[sl] / (sg + 1e-12)

            @pl.when(c == 0)
            def _():
                pltpu.sync_copy(w_v, out_hbm.at[pl.ds(off, _C)])
            return carry

        lax.fori_loop(0, n_chunks, ph2, 0)

    return k(s1, table, idx, zeros)


def _seg_sum32(vals_a, vals_b, idx, zeros32, m_sz, k_sz):
    per_sub = m_sz // _NS
    n_chunks = per_sub // _C
    rows_out = k_sz // _NS

    @functools.partial(
        pl.kernel,
        mesh=_MESH,
        out_type=[jax.ShapeDtypeStruct((k_sz, H // 2), _f32),
                  jax.ShapeDtypeStruct((k_sz, H // 2), _f32)],
        scratch_types=[
            pltpu.VMEM_SHARED((k_sz, H // 2), _f32),
            pltpu.VMEM((1, _C), jnp.int32),
            pltpu.VMEM((_C, H // 2), _f32),
        ],
        compiler_params=_SC_PARAMS,
    )
    def k(va_hbm, vb_hbm, idx_hbm, z_hbm, oa_hbm, ob_hbm,
          spacc, idx_v, vals_v):
        c = lax.axis_index("c")
        s = lax.axis_index("s")

        @pl.when(s == 0)
        def _():
            pltpu.sync_copy(z_hbm, spacc)

        plsc.subcore_barrier()

        def body(i, carry):
            off = s * per_sub + i * _C
            pltpu.sync_copy(idx_hbm.at[pl.ds(off, _C)], idx_v.at[0])

            @pl.when(c == 0)
            def _():
                pltpu.sync_copy(va_hbm.at[pl.ds(off, _C)], vals_v)

            @pl.when(c == 1)
            def _():
                pltpu.sync_copy(vb_hbm.at[pl.ds(off, _C)], vals_v)

            pltpu.sync_copy(vals_v, spacc.at[idx_v.at[0]], add=True)
            return carry

        lax.fori_loop(0, n_chunks, body, 0)
        plsc.subcore_barrier()
        o = s * rows_out

        @pl.when(c == 0)
        def _():
            pltpu.sync_copy(spacc.at[pl.ds(o, rows_out)],
                            oa_hbm.at[pl.ds(o, rows_out)])

        @pl.when(c == 1)
        def _():
            pltpu.sync_copy(spacc.at[pl.ds(o, rows_out)],
                            ob_hbm.at[pl.ds(o, rows_out)])

    return k(vals_a, vals_b, idx, zeros32)


def kernel(node, edge, edge_index, node_graph_ids, params):
    f32 = _f32
    node_p = jnp.zeros((N_PAD, H), f32).at[:N, :node.shape[1]].set(node)
    edge_p = jnp.zeros((E_PAD, 16), f32).at[:E, :edge.shape[1]].set(edge)
    src = jnp.clip(edge_index[0].astype(jnp.int32), 0, N - 1)
    src_p = jnp.zeros((E_PAD,), jnp.int32).at[:E].set(src)
    dst_p = jnp.full((E_PAD,), N, jnp.int32).at[:E].set(
        edge_index[1].astype(jnp.int32))
    gids_p = jnp.full((N_PAD,), G, jnp.int32).at[:N].set(
        node_graph_ids.astype(jnp.int32))

    zN = jnp.zeros((N_PAD,), f32)
    zN32 = jnp.zeros((N_PAD, H // 2), f32)
    zG = jnp.zeros((G_PAD,), f32)
    zG32 = jnp.zeros((G_PAD, H // 2), f32)

    embN_W = jnp.zeros((H, H), f32).at[:, :node.shape[1]].set(params['embN_W'])
    embE_W = jnp.zeros((H, 16), f32).at[:, :edge.shape[1]].set(params['embE_W'])

    x = _tc_embed(node_p, embN_W, params['embN_b'].reshape(1, H), _BN)
    e = _tc_embed(edge_p, embE_W, params['embE_b'].reshape(1, H), _BE)

    for p in params['atom']:
        w1 = p['edge_W'][:, :H]
        w2 = p['edge_W'][:, H:]
        a1 = p['align_W'][:, :H]
        a2 = p['align_W'][:, H:]
        ab = p['align_b'].reshape(1, 1)
        h, hW1, hda = _tc_node(x, p['node_W'], p['node_b'].reshape(1, H),
                               w1, a2)
        hsrc = _gather_rows(hW1, src_p)
        nm, s1 = _tc_edge(hsrc, e, w2, p['edge_b'].reshape(1, H), a1, ab)
        att = _seg_softmax_sc(s1.reshape(E_PAD), hda.reshape(N_PAD), dst_p,
                              zN, E_PAD, N_PAD)
        attcA, attcB = _tc_attc(nm, att.reshape(E_PAD, 1), p['attend_W'],
                                p['attend_b'].reshape(1, H), _BE)
        cA, cB = _seg_sum32(attcA, attcB, dst_p, zN32, E_PAD, N_PAD)
        x = _tc_gru(cA, cB, h, p['gru'], _BN)

    xA, xB = _tc_split(x)
    sA, sB = _seg_sum32(xA, xB, gids_p, zG32, N_PAD, G_PAD)
    s = jnp.concatenate([sA, sB], axis=1)

    for p in params['mol']:
        a1 = p['align_W'][:, :H]
        a2 = p['align_W'][:, H:]
        ab = p['align_b'].reshape(1, 1)
        sa, ga = _tc_molg(s, a2)
        s1m, hn = _tc_moln(x, a1, ab, p['attend_W'],
                           p['attend_b'].reshape(1, H))
        aw = _seg_softmax_sc(s1m.reshape(N_PAD), ga.reshape(G_PAD), gids_p,
                             zG, N_PAD, G_PAD)
        awhnA, awhnB = _tc_awhn(hn, aw.reshape(N_PAD, 1))
        cA, cB = _seg_sum32(awhnA, awhnB, gids_p, zG32, N_PAD, G_PAD)
        s = _tc_gru(cA, cB, sa, p['gru'], G_PAD)

    out = _tc_pred(s, params['pred_W'], params['pred_b'].reshape(1, 1))
    return out[:G]


# speedup vs baseline: 4.6685x; 1.3342x over previous
import functools

import jax
import jax.numpy as jnp
from jax import lax
from jax.experimental import pallas as pl
from jax.experimental.pallas import tpu as pltpu
from jax.experimental.pallas import tpu_sc as plsc

N = 50000
E = 800000
G = 2000
H = 64

N_PAD = 51200
E_PAD = 819200
G_PAD = 2048

_BN = 3200
_BE = 4096

_f32 = jnp.float32


def _lrelu(x):
    return jnp.where(x >= 0, x, 0.01 * x)


def _dot_t(x, w):
    return lax.dot_general(x, w, (((1,), (1,)), ((), ())),
                           preferred_element_type=_f32)


def _rowdot(x, a):
    return jnp.sum(x * a, axis=1, keepdims=True)


def _row_spec(bm, n):
    return pl.BlockSpec((bm, n), lambda i: (i, 0))


def _full_spec(a, b):
    return pl.BlockSpec((a, b), lambda i: (0, 0))


def _emb_body(x_ref, w_ref, b_ref, o_ref):
    o_ref[...] = _lrelu(_dot_t(x_ref[...], w_ref[...]) + b_ref[...])


def _tc_embed(x, w, b, bm):
    m, k = x.shape
    h = w.shape[0]
    return pl.pallas_call(
        _emb_body,
        grid=(m // bm,),
        in_specs=[_row_spec(bm, k), _full_spec(h, k), _full_spec(1, h)],
        out_specs=_row_spec(bm, h),
        out_shape=jax.ShapeDtypeStruct((m, h), _f32),
    )(x, w, b)


def _node_body(x_ref, wn_ref, bn_ref, w1_ref, a2_ref, h_ref, hw1_ref, hda_ref):
    h = _lrelu(_dot_t(x_ref[...], wn_ref[...]) + bn_ref[...])
    h_ref[...] = h
    hw1_ref[...] = _dot_t(h, w1_ref[...])
    hda_ref[...] = _rowdot(h, a2_ref[...])


def _tc_node(x, wn, bn, w1, a2):
    return pl.pallas_call(
        _node_body,
        grid=(N_PAD // _BN,),
        in_specs=[_row_spec(_BN, H), _full_spec(H, H), _full_spec(1, H),
                  _full_spec(H, H), _full_spec(1, H)],
        out_specs=[_row_spec(_BN, H), _row_spec(_BN, H), _row_spec(_BN, 1)],
        out_shape=[jax.ShapeDtypeStruct((N_PAD, H), _f32),
                   jax.ShapeDtypeStruct((N_PAD, H), _f32),
                   jax.ShapeDtypeStruct((N_PAD, 1), _f32)],
    )(x, wn, bn, w1, a2)


def _edge_body(hs_ref, e_ref, w2_ref, eb_ref, a1_ref, ab_ref, nm_ref, s1_ref):
    nm = _lrelu(hs_ref[...] + _dot_t(e_ref[...], w2_ref[...]) + eb_ref[...])
    nm_ref[...] = nm
    s1_ref[...] = _rowdot(nm, a1_ref[...]) + ab_ref[0, 0]


def _tc_edge(hsrc, e, w2, eb, a1, ab):
    return pl.pallas_call(
        _edge_body,
        grid=(E_PAD // _BE,),
        in_specs=[_row_spec(_BE, H), _row_spec(_BE, H), _full_spec(H, H),
                  _full_spec(1, H), _full_spec(1, H), _full_spec(1, 1)],
        out_specs=[_row_spec(_BE, H), _row_spec(_BE, 1)],
        out_shape=[jax.ShapeDtypeStruct((E_PAD, H), _f32),
                   jax.ShapeDtypeStruct((E_PAD, 1), _f32)],
    )(hsrc, e, w2, eb, a1, ab)


def _attc_body(nm_ref, att_ref, w_ref, b_ref, oa_ref, ob_ref):
    attn = _dot_t(nm_ref[...], w_ref[...]) + b_ref[...]
    attc = att_ref[...] * attn
    oa_ref[...] = attc[:, :H // 2]
    ob_ref[...] = attc[:, H // 2:]


def _tc_attc(nm, att, w, b, bm):
    m = nm.shape[0]
    return pl.pallas_call(
        _attc_body,
        grid=(m // bm,),
        in_specs=[_row_spec(bm, H), _row_spec(bm, 1), _full_spec(H, H),
                  _full_spec(1, H)],
        out_specs=[_row_spec(bm, H // 2), _row_spec(bm, H // 2)],
        out_shape=[jax.ShapeDtypeStruct((m, H // 2), _f32),
                   jax.ShapeDtypeStruct((m, H // 2), _f32)],
    )(nm, att, w, b)


def _gru_body(ca_ref, cb_ref, h_ref, wih_ref, whh_ref, bih_ref, bhh_ref,
              o_ref):
    ctx = jnp.concatenate([ca_ref[...], cb_ref[...]], axis=1)
    ctx = jnp.where(ctx > 0, ctx, jnp.exp(ctx) - 1.0)
    h = h_ref[...]
    gi = _dot_t(ctx, wih_ref[...]) + bih_ref[...]
    gh = _dot_t(h, whh_ref[...]) + bhh_ref[...]
    r = jax.nn.sigmoid(gi[:, :H] + gh[:, :H])
    z = jax.nn.sigmoid(gi[:, H:2 * H] + gh[:, H:2 * H])
    n = jnp.tanh(gi[:, 2 * H:] + r * gh[:, 2 * H:])
    o_ref[...] = jnp.maximum((1.0 - z) * n + z * h, 0.0)


def _tc_gru(ca, cb, h, g, bm):
    m = ca.shape[0]
    return pl.pallas_call(
        _gru_body,
        grid=(m // bm,),
        in_specs=[_row_spec(bm, H // 2), _row_spec(bm, H // 2),
                  _row_spec(bm, H), _full_spec(3 * H, H), _full_spec(3 * H, H),
                  _full_spec(1, 3 * H), _full_spec(1, 3 * H)],
        out_specs=_row_spec(bm, H),
        out_shape=jax.ShapeDtypeStruct((m, H), _f32),
    )(ca, cb, h, g['Wih'], g['Whh'], g['bih'].reshape(1, -1),
      g['bhh'].reshape(1, -1))


def _split_body(x_ref, oa_ref, ob_ref):
    x = x_ref[...]
    oa_ref[...] = x[:, :H // 2]
    ob_ref[...] = x[:, H // 2:]


def _tc_split(x):
    return pl.pallas_call(
        _split_body,
        grid=(N_PAD // _BN,),
        in_specs=[_row_spec(_BN, H)],
        out_specs=[_row_spec(_BN, H // 2), _row_spec(_BN, H // 2)],
        out_shape=[jax.ShapeDtypeStruct((N_PAD, H // 2), _f32),
                   jax.ShapeDtypeStruct((N_PAD, H // 2), _f32)],
    )(x)


def _molg_body(s_ref, a2_ref, sa_ref, ga_ref):
    sa = _lrelu(s_ref[...])
    sa_ref[...] = sa
    ga_ref[...] = _rowdot(sa, a2_ref[...])


def _tc_molg(s, a2):
    return pl.pallas_call(
        _molg_body,
        grid=(1,),
        in_specs=[_row_spec(G_PAD, H), _full_spec(1, H)],
        out_specs=[_row_spec(G_PAD, H), _row_spec(G_PAD, 1)],
        out_shape=[jax.ShapeDtypeStruct((G_PAD, H), _f32),
                   jax.ShapeDtypeStruct((G_PAD, 1), _f32)],
    )(s, a2)


def _moln_body(x_ref, a1_ref, ab_ref, w_ref, b_ref, s1_ref, hn_ref):
    x = x_ref[...]
    s1_ref[...] = _rowdot(x, a1_ref[...]) + ab_ref[0, 0]
    hn_ref[...] = _dot_t(x, w_ref[...]) + b_ref[...]


def _tc_moln(x, a1, ab, w, b):
    return pl.pallas_call(
        _moln_body,
        grid=(N_PAD // _BN,),
        in_specs=[_row_spec(_BN, H), _full_spec(1, H), _full_spec(1, 1),
                  _full_spec(H, H), _full_spec(1, H)],
        out_specs=[_row_spec(_BN, 1), _row_spec(_BN, H)],
        out_shape=[jax.ShapeDtypeStruct((N_PAD, 1), _f32),
                   jax.ShapeDtypeStruct((N_PAD, H), _f32)],
    )(x, a1, ab, w, b)


def _awhn_body(hn_ref, aw_ref, oa_ref, ob_ref):
    v = aw_ref[...] * hn_ref[...]
    oa_ref[...] = v[:, :H // 2]
    ob_ref[...] = v[:, H // 2:]


def _tc_awhn(hn, aw):
    return pl.pallas_call(
        _awhn_body,
        grid=(N_PAD // _BN,),
        in_specs=[_row_spec(_BN, H), _row_spec(_BN, 1)],
        out_specs=[_row_spec(_BN, H // 2), _row_spec(_BN, H // 2)],
        out_shape=[jax.ShapeDtypeStruct((N_PAD, H // 2), _f32),
                   jax.ShapeDtypeStruct((N_PAD, H // 2), _f32)],
    )(hn, aw)


def _pred_body(s_ref, w_ref, b_ref, o_ref):
    o_ref[...] = _rowdot(s_ref[...], w_ref[...]) + b_ref[0, 0]


def _tc_pred(s, w, b):
    return pl.pallas_call(
        _pred_body,
        grid=(1,),
        in_specs=[_row_spec(G_PAD, H), _full_spec(1, H), _full_spec(1, 1)],
        out_specs=_row_spec(G_PAD, 1),
        out_shape=jax.ShapeDtypeStruct((G_PAD, 1), _f32),
    )(s, w, b)


_MESH = plsc.VectorSubcoreMesh(core_axis_name="c", subcore_axis_name="s")
_SC_PARAMS = pltpu.CompilerParams(use_tc_tiling_on_sc=False,
                                  needs_layout_passes=False)
_NC = 2
_NS = 16
_NW = _NC * _NS
_C = 128
_GR = 512


def _gather_rows(table, idx2):
    rows = _GR // _C
    per_w = E_PAD // _NW
    n_chunks = per_w // _GR
    n_pairs = n_chunks // 2

    @functools.partial(
        pl.kernel,
        mesh=_MESH,
        out_type=jax.ShapeDtypeStruct((E_PAD // _C, _C, H), _f32),
        scratch_types=[
            pltpu.VMEM((rows, _C), jnp.int32),
            pltpu.VMEM((rows, _C), jnp.int32),
            pltpu.VMEM((rows, _C, H), _f32),
            pltpu.VMEM((rows, _C, H), _f32),
            pltpu.SemaphoreType.DMA,
            pltpu.SemaphoreType.DMA,
        ],
        compiler_params=_SC_PARAMS,
    )
    def k(table_hbm, idx_hbm, out_hbm, i0, i1, r0, r1, s0, s1):
        wid = lax.axis_index("s") * _NC + lax.axis_index("c")
        ibase = wid * (per_w // _C)
        idx_b = (i0, i1)
        row_b = (r0, r1)
        sem_b = (s0, s1)

        def fire(b, crow):
            pltpu.sync_copy(idx_hbm.at[pl.ds(crow, rows)], idx_b[b])
            for j in range(rows):
                pltpu.async_copy(table_hbm.at[idx_b[b].at[j]],
                                 row_b[b].at[j], sem_b[b])

        def drain(b):
            for j in range(rows):
                pltpu.make_async_copy(table_hbm.at[idx_b[b].at[j]],
                                      row_b[b].at[j], sem_b[b]).wait()

        for b in range(2):
            fire(b, ibase + b * rows)

        def body(i, carry):
            for b in range(2):
                c = 2 * i + b
                crow = ibase + c * rows
                drain(b)
                pltpu.sync_copy(row_b[b], out_hbm.at[pl.ds(crow, rows)])

                @pl.when(c + 2 < n_chunks)
                def _():
                    fire(b, crow + 2 * rows)
            return carry

        lax.fori_loop(0, n_pairs, body, 0)

    return k(table, idx2)


def _seg_softmax_sc(s1_2d, table, idx2, zeros, m_sz, k_sz, rows):
    m_rows = m_sz // _C
    per_sub = m_rows // _NS
    n_chunks = per_sub // rows

    @functools.partial(
        pl.kernel,
        mesh=_MESH,
        out_type=jax.ShapeDtypeStruct((m_rows, _C), _f32),
        scratch_types=[
            pltpu.VMEM_SHARED((k_sz,), _f32),
            pltpu.VMEM((k_sz,), _f32),
            pltpu.VMEM((k_sz,), _f32),
            pltpu.VMEM((rows, _C), jnp.int32),
            pltpu.VMEM((rows, _C), _f32),
            pltpu.VMEM((rows, _C), _f32),
            pltpu.SemaphoreType.DMA,
        ],
        compiler_params=_SC_PARAMS,
    )
    def k(s1_hbm, tab_hbm, idx_hbm, z_hbm, out_hbm,
          spsum, tab_v, sums_v, idx_v, s1_v, w_v, sem):
        c = lax.axis_index("c")
        s = lax.axis_index("s")
        pltpu.sync_copy(tab_hbm, tab_v)

        @pl.when(s == 0)
        def _():
            pltpu.sync_copy(z_hbm, spsum)

        plsc.subcore_barrier()

        def _chunk_w(roff):
            pltpu.sync_copy(idx_hbm.at[pl.ds(roff, rows)], idx_v)
            pltpu.sync_copy(s1_hbm.at[pl.ds(roff, rows)], s1_v)
            for j in range(rows):
                for q in range(_C // 16):
                    sl = pl.ds(q * 16, 16)
                    iv = idx_v[j, sl]
                    tv = plsc.load_gather(tab_v, [iv])
                    sc = s1_v[j, sl] + tv
                    sc = jnp.where(sc >= 0, sc, 0.01 * sc)
                    w_v[j, sl] = jnp.exp(sc)

        def ph1(i, carry):
            roff = s * per_sub + i * rows
            _chunk_w(roff)
            for j in range(rows):
                pltpu.async_copy(w_v.at[j], spsum.at[idx_v.at[j]], sem,
                                 add=True)
            for j in range(rows):
                pltpu.make_async_copy(w_v.at[j], spsum.at[idx_v.at[j]],
                                      sem).wait()
            return carry

        lax.fori_loop(0, n_chunks, ph1, 0)
        plsc.subcore_barrier()
        pltpu.sync_copy(spsum, sums_v)

        def ph2(i, carry):
            roff = s * per_sub + (2 * i + c) * rows
            _chunk_w(roff)
            for j in range(rows):
                for q in range(_C // 16):
                    sl = pl.ds(q * 16, 16)
                    iv = idx_v[j, sl]
                    sg = plsc.load_gather(sums_v, [iv])
                    w_v[j, sl] = w_v[j, sl] / (sg + 1e-12)
            pltpu.sync_copy(w_v, out_hbm.at[pl.ds(roff, rows)])
            return carry

        lax.fori_loop(0, (n_chunks + 1 - c) // 2, ph2, 0)

    return k(s1_2d, table, idx2, zeros)


def _seg_sum32(vals_a, vals_b, idx2, zeros32, m_sz, k_sz, rows):
    chunk = rows * _C
    per_sub = m_sz // _NS
    n_chunks = per_sub // chunk
    rows_out = k_sz // _NS

    @functools.partial(
        pl.kernel,
        mesh=_MESH,
        out_type=[jax.ShapeDtypeStruct((k_sz, H // 2), _f32),
                  jax.ShapeDtypeStruct((k_sz, H // 2), _f32)],
        scratch_types=[
            pltpu.VMEM_SHARED((k_sz, H // 2), _f32),
            pltpu.VMEM((rows, _C), jnp.int32),
            pltpu.VMEM((rows, _C, H // 2), _f32),
            pltpu.SemaphoreType.DMA,
        ],
        compiler_params=_SC_PARAMS,
    )
    def k(va_hbm, vb_hbm, idx_hbm, z_hbm, oa_hbm, ob_hbm,
          spacc, idx_v, vals_v, sem):
        c = lax.axis_index("c")
        s = lax.axis_index("s")

        @pl.when(s == 0)
        def _():
            pltpu.sync_copy(z_hbm, spacc)

        plsc.subcore_barrier()

        def body(i, carry):
            roff = s * (per_sub // _C) + i * rows
            pltpu.sync_copy(idx_hbm.at[pl.ds(roff, rows)], idx_v)

            @pl.when(c == 0)
            def _():
                pltpu.sync_copy(va_hbm.at[pl.ds(roff, rows)], vals_v)

            @pl.when(c == 1)
            def _():
                pltpu.sync_copy(vb_hbm.at[pl.ds(roff, rows)], vals_v)

            for j in range(rows):
                pltpu.async_copy(vals_v.at[j], spacc.at[idx_v.at[j]], sem,
                                 add=True)
            for j in range(rows):
                pltpu.make_async_copy(vals_v.at[j], spacc.at[idx_v.at[j]],
                                      sem).wait()
            return carry

        lax.fori_loop(0, n_chunks, body, 0)
        plsc.subcore_barrier()
        o = s * rows_out

        @pl.when(c == 0)
        def _():
            pltpu.sync_copy(spacc.at[pl.ds(o, rows_out)],
                            oa_hbm.at[pl.ds(o, rows_out)])

        @pl.when(c == 1)
        def _():
            pltpu.sync_copy(spacc.at[pl.ds(o, rows_out)],
                            ob_hbm.at[pl.ds(o, rows_out)])

    return k(vals_a, vals_b, idx2, zeros32)


def kernel(node, edge, edge_index, node_graph_ids, params):
    f32 = _f32
    node_p = jnp.zeros((N_PAD, H), f32).at[:N, :node.shape[1]].set(node)
    edge_p = jnp.zeros((E_PAD, 16), f32).at[:E, :edge.shape[1]].set(edge)
    src = jnp.clip(edge_index[0].astype(jnp.int32), 0, N - 1)
    src2 = jnp.zeros((E_PAD,), jnp.int32).at[:E].set(src).reshape(
        E_PAD // _C, _C)
    dst2 = jnp.full((E_PAD,), N, jnp.int32).at[:E].set(
        edge_index[1].astype(jnp.int32)).reshape(E_PAD // _C, _C)
    gids2 = jnp.full((N_PAD,), G, jnp.int32).at[:N].set(
        node_graph_ids.astype(jnp.int32)).reshape(N_PAD // _C, _C)

    zN = jnp.zeros((N_PAD,), f32)
    zN32 = jnp.zeros((N_PAD, H // 2), f32)
    zG = jnp.zeros((G_PAD,), f32)
    zG32 = jnp.zeros((G_PAD, H // 2), f32)

    embN_W = jnp.zeros((H, H), f32).at[:, :node.shape[1]].set(params['embN_W'])
    embE_W = jnp.zeros((H, 16), f32).at[:, :edge.shape[1]].set(params['embE_W'])

    x = _tc_embed(node_p, embN_W, params['embN_b'].reshape(1, H), _BN)
    e = _tc_embed(edge_p, embE_W, params['embE_b'].reshape(1, H), _BE)

    for p in params['atom']:
        w1 = p['edge_W'][:, :H]
        w2 = p['edge_W'][:, H:]
        a1 = p['align_W'][:, :H]
        a2 = p['align_W'][:, H:]
        ab = p['align_b'].reshape(1, 1)
        h, hW1, hda = _tc_node(x, p['node_W'], p['node_b'].reshape(1, H),
                               w1, a2)
        hsrc = _gather_rows(hW1, src2).reshape(E_PAD, H)
        nm, s1 = _tc_edge(hsrc, e, w2, p['edge_b'].reshape(1, H), a1, ab)
        att = _seg_softmax_sc(s1.reshape(E_PAD // _C, _C),
                              hda.reshape(N_PAD), dst2, zN, E_PAD, N_PAD, 8)
        attcA, attcB = _tc_attc(nm, att.reshape(E_PAD, 1), p['attend_W'],
                                p['attend_b'].reshape(1, H), _BE)
        cA, cB = _seg_sum32(attcA.reshape(E_PAD // _C, _C, H // 2),
                            attcB.reshape(E_PAD // _C, _C, H // 2),
                            dst2, zN32, E_PAD, N_PAD, 4)
        x = _tc_gru(cA, cB, h, p['gru'], _BN)

    xA, xB = _tc_split(x)
    sA, sB = _seg_sum32(xA.reshape(N_PAD // _C, _C, H // 2),
                        xB.reshape(N_PAD // _C, _C, H // 2),
                        gids2, zG32, N_PAD, G_PAD, 25)
    s = jnp.concatenate([sA, sB], axis=1)

    for p in params['mol']:
        a1 = p['align_W'][:, :H]
        a2 = p['align_W'][:, H:]
        ab = p['align_b'].reshape(1, 1)
        sa, ga = _tc_molg(s, a2)
        s1m, hn = _tc_moln(x, a1, ab, p['attend_W'],
                           p['attend_b'].reshape(1, H))
        aw = _seg_softmax_sc(s1m.reshape(N_PAD // _C, _C),
                             ga.reshape(G_PAD), gids2, zG, N_PAD, G_PAD, 5)
        awhnA, awhnB = _tc_awhn(hn, aw.reshape(N_PAD, 1))
        cA, cB = _seg_sum32(awhnA.reshape(N_PAD // _C, _C, H // 2),
                            awhnB.reshape(N_PAD // _C, _C, H // 2),
                            gids2, zG32, N_PAD, G_PAD, 25)
        s = _tc_gru(cA, cB, sa, p['gru'], G_PAD)

    out = _tc_pred(s, params['pred_W'], params['pred_b'].reshape(1, 1))
    return out[:G]
